# baseline (speedup 1.0000x reference)
import functools

import jax
import jax.numpy as jnp
from jax import lax
from jax.experimental import pallas as pl
from jax.experimental.pallas import tpu as pltpu
from jax.experimental.pallas import tpu_sc as plsc

_T = 2048
_LANES = 16
_NCH = 64
_NIN = 40


@functools.lru_cache(maxsize=None)
def _lr_kernel(B, L, D):
    T = _T
    n_chunks = T // _NCH
    chunks_per_core = n_chunks // 2
    vregs_per_chunk = _NCH // _LANES
    mesh = plsc.VectorSubcoreMesh(core_axis_name="c", subcore_axis_name="s")

    @functools.partial(
        pl.kernel,
        out_type=jax.ShapeDtypeStruct((B, T, D), jnp.float32),
        mesh=mesh,
        compiler_params=pltpu.CompilerParams(needs_layout_passes=False),
        scratch_types=[
            pltpu.VMEM((L,), jnp.float32),
            pltpu.VMEM((T,), jnp.int32),
            pltpu.VMEM((n_chunks, _NCH), jnp.int32),
            pltpu.VMEM((n_chunks, _NCH), jnp.int32),
            pltpu.VMEM((_LANES,), jnp.int32),
            pltpu.VMEM((_NIN + 1, D), jnp.float32),
            pltpu.VMEM((_NIN + 1, D), jnp.float32),
            pltpu.VMEM((_NCH, D), jnp.float32),
            pltpu.VMEM((_NCH, D), jnp.float32),
            pltpu.VMEM((_NCH, D), jnp.float32),
            pltpu.SMEM((n_chunks,), jnp.int32),
            pltpu.SMEM((n_chunks,), jnp.int32),
            pltpu.SemaphoreType.DMA,
            pltpu.SemaphoreType.DMA,
            pltpu.SemaphoreType.DMA,
            pltpu.SemaphoreType.DMA,
            pltpu.SemaphoreType.DMA,
        ],
    )
    def k(table_hbm, dur_hbm, off_hbm, out_hbm,
          dur_v, seg_v, src_v, gid_v, off_v, in_a, in_b, out_a, out_b,
          zero_v, lo_s, span_s, ls_a, ls_b, ws_a, ws_b, ssem):
        b = lax.axis_index("s")
        h = lax.axis_index("c")
        pltpu.sync_copy(dur_hbm.at[b], dur_v)
        pltpu.sync_copy(off_hbm, off_v)
        off = off_v[...]
        off_sc = jnp.max(off)

        def zero_body(i, _):
            seg_v[pl.ds(i * _LANES, _LANES)] = jnp.zeros((_LANES,), jnp.int32)
            return 0

        lax.fori_loop(0, T // _LANES, zero_body, 0)
        for v in (in_a, in_b):
            for j in range(D // _LANES):
                v[_NIN, pl.ds(j * _LANES, _LANES)] = jnp.zeros(
                    (_LANES,), jnp.float32)

        def zchunk_body(t, _):
            for j in range(D // _LANES):
                zero_v[t, pl.ds(j * _LANES, _LANES)] = jnp.zeros(
                    (_LANES,), jnp.float32)
            return 0

        lax.fori_loop(0, _NCH, zchunk_body, 0)

        def scat_body(i, carry):
            tot, basev = carry
            dv = jnp.maximum(dur_v[pl.ds(i * _LANES, _LANES)], 0.0)
            di = (dv + 0.5).astype(jnp.int32)
            cum = plsc.cumsum(di) + tot
            pos = cum - di - off
            lv = lax.iota(jnp.int32, _LANES) + i * _LANES + 1
            valid = di > 0
            m = valid & (pos >= 0) & (pos < T)
            plsc.store_scatter(seg_v, [jnp.clip(pos, 0, T - 1)], lv, mask=m)
            basev = jnp.maximum(basev, jnp.where(valid & (pos < 0), lv, 0))
            return jnp.max(cum), basev

        total, basev = lax.fori_loop(
            0, L // _LANES, scat_body,
            (jnp.asarray(0, jnp.int32), jnp.zeros((_LANES,), jnp.int32)))
        base = jnp.max(basev)

        def chunk_idx_body(r, mc0):
            def q_body(q, carry):
                mc, cl, hik = carry
                i = r * vregs_per_chunk + q
                s = seg_v[pl.ds(i * _LANES, _LANES)]
                cm = jnp.maximum(plsc.cummax(s), mc)
                idx = jnp.clip(cm - 1, 0, L - 1)
                cl = jnp.where(
                    q == 0,
                    jnp.minimum((jnp.min(idx) // 8) * 8, L - _NIN), cl)
                kv = lax.iota(jnp.int32, _LANES) + i * _LANES
                keep = kv + off < total
                hik = jnp.maximum(hik, jnp.max(jnp.where(keep, idx, 0)))
                src_v[r, pl.ds(q * _LANES, _LANES)] = jnp.where(
                    keep, idx - cl, _NIN)
                return jnp.max(cm), cl, hik

            mc, cl, hik = lax.fori_loop(
                0, vregs_per_chunk, q_body,
                (mc0, jnp.asarray(0, jnp.int32), jnp.asarray(0, jnp.int32)))
            lo_s[r] = cl
            span_s[r] = hik - cl
            return mc

        mc4 = lax.fori_loop(0, 4, chunk_idx_body, base)

        ins = (in_a, in_b)
        outs = (out_a, out_b)
        lsems = (ls_a, ls_b)
        wsems = (ws_a, ws_b)

        def is_pad(r):
            return total - off_sc - r * _NCH <= 0

        def load_slice(r):
            return table_hbm.at[
                pl.ds(pl.multiple_of(b * L + lo_s[r], 8), _NIN), :]

        def out_slice(r):
            return out_hbm.at[b, pl.ds(r * _NCH, _NCH), :]

        def expand(r, in_v, out_v, g0, g1):
            @plsc.parallel_loop(g0, g1)
            def g_body(g):
                base = g * _LANES
                srcv = src_v[r, pl.ds(base, _LANES)]
                ss = [srcv[c] for c in range(_LANES)]
                nil = 8
                for c in range(0, _LANES, nil):
                    t = base + c
                    for j in range(D // _LANES):
                        vs = [in_v[ss[c + u], pl.ds(j * _LANES, _LANES)]
                              for u in range(nil)]
                        for u in range(nil):
                            out_v[t + u, pl.ds(j * _LANES, _LANES)] = vs[u]

        def gather_fallback(r, out_v):
            def gid_body(q, _):
                sv = src_v[r, pl.ds(q * _LANES, _LANES)]
                gid_v[r, pl.ds(q * _LANES, _LANES)] = (
                    b * L + jnp.minimum(sv + lo_s[r], L - 1))
                return 0

            lax.fori_loop(0, vregs_per_chunk, gid_body, 0)
            pltpu.async_copy(table_hbm.at[gid_v.at[r]], out_v, ssem).wait()
            klim = jnp.clip(total - off_sc - r * _NCH, 0, _NCH)

            def z_body(t, _):
                @pl.when(t >= klim)
                def _():
                    for j in range(D // _LANES):
                        out_v[t, pl.ds(j * _LANES, _LANES)] = jnp.zeros(
                            (_LANES,), jnp.float32)
                return 0

            lax.fori_loop(0, _NCH, z_body, 0)

        def fire_load(r, p):
            @pl.when(jnp.logical_not(is_pad(r)))
            def _():
                pltpu.async_copy(
                    load_slice(r), ins[p].at[pl.ds(0, _NIN), :], lsems[p])

        r0 = h
        fire_load(r0, 0)
        fire_load(r0 + 2, 1)
        lax.fori_loop(4, n_chunks, chunk_idx_body, mc4)

        def chunk_body(i, _):
            for p in (0, 1):
                cix = 2 * i + p
                r = 2 * cix + h
                in_v, out_v = ins[p], outs[p]
                pad = is_pad(r)
                live = jnp.logical_not(pad)

                @pl.when(live)
                def _():
                    pltpu.make_async_copy(
                        load_slice(r), in_v.at[pl.ds(0, _NIN), :],
                        lsems[p]).wait()

                @pl.when(cix >= 2)
                def _():
                    pltpu.make_async_copy(out_v, out_slice(r), wsems[p]).wait()

                @pl.when(live & (span_s[r] <= _NIN - 1))
                def _():
                    expand(r, in_v, out_v, 0, vregs_per_chunk)

                @pl.when(live & (span_s[r] > _NIN - 1))
                def _():
                    gather_fallback(r, out_v)

                @pl.when(live)
                def _():
                    pltpu.async_copy(out_v, out_slice(r), wsems[p])

                @pl.when(pad)
                def _():
                    pltpu.async_copy(zero_v, out_slice(r), wsems[p])

                @pl.when(cix + 2 < chunks_per_core)
                def _():
                    fire_load(r + 4, p)
            return 0

        lax.fori_loop(0, chunks_per_core // 2, chunk_body, 0)
        pltpu.make_async_copy(outs[0], out_slice(0), wsems[0]).wait()
        pltpu.make_async_copy(outs[1], out_slice(0), wsems[1]).wait()

    return k


def kernel(x, durations, max_len):
    B, L, D = x.shape
    table = x.reshape(B * L, D)
    off = jnp.full((_LANES,), jnp.asarray(max_len, jnp.int32) - _T, jnp.int32)
    return _lr_kernel(B, L, D)(table, durations, off)
